# baseline (speedup 1.0000x reference)
import functools

import jax
import jax.numpy as jnp
from jax import lax
from jax.experimental import pallas as pl
from jax.experimental.pallas import tpu as pltpu
from jax.experimental.pallas import tpu_sc as plsc

_N = 10000
_E = 320000
_D = 128
_NP = 10240
_NC = 2
_NS = 16
_NW = _NC * _NS
_EB = _E // 128
_EBUF = 79 * 128
_CSL = _NP // _NS
_L = 16
_UNROLL = 8

_mesh = plsc.VectorSubcoreMesh(core_axis_name="c", subcore_axis_name="s")
_sc_params = pltpu.CompilerParams(needs_layout_passes=False)


def _edge_range(wid):
    blk_s = (_EB * wid) // _NW
    blk_e = (_EB * (wid + 1)) // _NW
    base = pl.multiple_of(blk_s * 128, 128)
    nvreg = (blk_e - blk_s) * (128 // _L)
    return base, nvreg


def _zero_vmem(ref, n):
    z = jnp.zeros((_L,), jnp.float32)

    def body(i, carry):
        for u in range(8):
            ref[pl.ds((i * 8 + u) * _L, _L)] = z
        return carry

    lax.fori_loop(0, n // (8 * _L), body, 0)


def _rsqrt_newton(d):
    i = plsc.bitcast(d, jnp.int32)
    i = jnp.full((_L,), 0x5F3759DF, jnp.int32) - (i >> 1)
    r = plsc.bitcast(i, jnp.float32)
    for _ in range(3):
        r = r * (1.5 - 0.5 * d * r * r)
    return r


@functools.partial(
    pl.kernel,
    out_type=jax.ShapeDtypeStruct((_NW, _NP), jnp.float32),
    mesh=_mesh,
    scratch_types=[
        pltpu.VMEM((2, _EBUF), jnp.int32),
        pltpu.VMEM((_NP,), jnp.float32),
        pltpu.SemaphoreType.DMA,
    ],
    compiler_params=_sc_params,
)
def _hist(edge_hbm, part_hbm, e_v, hist_v, sem):
    cid = lax.axis_index("c")
    sid = lax.axis_index("s")
    wid = sid * _NC + cid
    base, nvreg = _edge_range(wid)
    cp = pltpu.async_copy(edge_hbm.at[:, pl.ds(base, _EBUF)], e_v, sem)
    _zero_vmem(hist_v, _NP)
    cp.wait()
    one = jnp.ones((_L,), jnp.float32)

    def body(i, carry):
        for u in range(_UNROLL):
            idx = e_v[1, pl.ds((i * _UNROLL + u) * _L, _L)]
            plsc.addupdate_scatter(hist_v, [idx], one)
        return carry

    lax.fori_loop(0, nvreg // _UNROLL, body, 0)
    pltpu.sync_copy(hist_v, part_hbm.at[wid])


@functools.partial(
    pl.kernel,
    out_type=jax.ShapeDtypeStruct((_NW, _NP), jnp.float32),
    mesh=_mesh,
    scratch_types=[
        pltpu.VMEM((2, _EBUF), jnp.int32),
        pltpu.VMEM((_NP,), jnp.float32),
        pltpu.VMEM((_NP,), jnp.float32),
        pltpu.VMEM((_NW, _CSL), jnp.float32),
        pltpu.VMEM((_CSL,), jnp.float32),
        pltpu.VMEM((_CSL,), jnp.float32),
        pltpu.VMEM_SHARED((_NP,), jnp.float32),
        pltpu.SemaphoreType.DMA,
        pltpu.SemaphoreType.DMA,
    ],
    compiler_params=_sc_params,
)
def _edge_scatter(edge_hbm, degp_hbm, y_hbm, part_hbm, e_v, g_v, acc_v,
                  degp_v, y_v, gs_v, g_sh, sem, sem2):
    cid = lax.axis_index("c")
    sid = lax.axis_index("s")
    wid = sid * _NC + cid
    base, nvreg = _edge_range(wid)
    cp_e = pltpu.async_copy(edge_hbm.at[:, pl.ds(base, _EBUF)], e_v, sem)
    cp_ds = [
        pltpu.async_copy(degp_hbm.at[r, pl.ds(sid * _CSL, _CSL)],
                         degp_v.at[r], sem2)
        for r in range(_NW)
    ]
    cp_y = pltpu.async_copy(y_hbm.at[pl.ds(sid * _CSL, _CSL)], y_v, sem2)
    _zero_vmem(acc_v, _NP)
    for cp in cp_ds:
        cp.wait()
    cp_y.wait()

    def gbody(j, carry):
        s = degp_v[0, pl.ds(j * _L, _L)]
        for r in range(1, _NW):
            s = s + degp_v[r, pl.ds(j * _L, _L)]
        dinv = _rsqrt_newton(s + 1.0)
        gs_v[pl.ds(j * _L, _L)] = dinv * y_v[pl.ds(j * _L, _L)]
        return carry

    lax.fori_loop(0, _CSL // _L, gbody, 0)
    pltpu.sync_copy(gs_v, g_sh.at[pl.ds(sid * _CSL, _CSL)])
    plsc.subcore_barrier()
    pltpu.sync_copy(g_sh, g_v)
    cp_e.wait()

    def body(i, carry):
        for u in range(_UNROLL):
            off = (i * _UNROLL + u) * _L
            sidx = e_v[0, pl.ds(off, _L)]
            didx = e_v[1, pl.ds(off, _L)]
            vals = plsc.load_gather(g_v, [sidx])
            plsc.addupdate_scatter(acc_v, [didx], vals)
        return carry

    lax.fori_loop(0, nvreg // _UNROLL, body, 0)
    pltpu.sync_copy(acc_v, part_hbm.at[wid])


def _matvec_body(x_ref, w_ref, y_ref):
    y = jnp.dot(x_ref[...], w_ref[...],
                preferred_element_type=jnp.float32)[:, 0]
    y_ref[...] = jnp.concatenate([y, jnp.zeros((_NP - _N,), jnp.float32)])


_matvec = pl.pallas_call(
    _matvec_body,
    out_shape=jax.ShapeDtypeStruct((_NP,), jnp.float32),
)


def _final_body(accp_ref, degp_ref, y_ref, out_ref):
    acc = jnp.sum(accp_ref[...], axis=0)
    deg = jnp.sum(degp_ref[...], axis=0) + 1.0
    dinv = lax.rsqrt(deg)
    out_ref[...] = dinv * (acc + dinv * y_ref[...])


_final = pl.pallas_call(
    _final_body,
    out_shape=jax.ShapeDtypeStruct((_NP,), jnp.float32),
)


def kernel(x, edge_index, W):
    y = _matvec(x, W)
    deg_part = _hist(edge_index)
    acc_part = _edge_scatter(edge_index, deg_part, y)
    out = _final(acc_part, deg_part, y)
    return out[:_N]


# speedup vs baseline: 1.0657x; 1.0657x over previous
import functools

import jax
import jax.numpy as jnp
from jax import lax
from jax.experimental import pallas as pl
from jax.experimental.pallas import tpu as pltpu
from jax.experimental.pallas import tpu_sc as plsc

_N = 10000
_E = 320000
_D = 128
_NP = 10240
_NC = 2
_NS = 16
_NW = _NC * _NS
_EB = _E // 128
_EBUF = 79 * 128
_L = 16
_UNROLL = 4

_mesh = plsc.VectorSubcoreMesh(core_axis_name="c", subcore_axis_name="s")
_sc_params = pltpu.CompilerParams(needs_layout_passes=False)


def _edge_range(wid):
    blk_s = (_EB * wid) // _NW
    blk_e = (_EB * (wid + 1)) // _NW
    base = pl.multiple_of(blk_s * 128, 128)
    nvreg = (blk_e - blk_s) * (128 // _L)
    return base, nvreg


def _zero_vmem(ref, n):
    z = jnp.zeros((_L,), jnp.float32)

    def body(i, carry):
        for u in range(8):
            ref[pl.ds((i * 8 + u) * _L, _L)] = z
        return carry

    lax.fori_loop(0, n // (8 * _L), body, 0)


@functools.partial(
    pl.kernel,
    out_type=jax.ShapeDtypeStruct((_NW, _NP), jnp.float32),
    mesh=_mesh,
    scratch_types=[
        pltpu.VMEM((2, _EBUF), jnp.int32),
        pltpu.VMEM((_NP,), jnp.float32),
        pltpu.SemaphoreType.DMA,
    ],
    compiler_params=_sc_params,
)
def _hist(edge_hbm, part_hbm, e_v, hist_v, sem):
    cid = lax.axis_index("c")
    sid = lax.axis_index("s")
    wid = sid * _NC + cid
    base, nvreg = _edge_range(wid)
    cp = pltpu.async_copy(edge_hbm.at[:, pl.ds(base, _EBUF)], e_v, sem)
    _zero_vmem(hist_v, _NP)
    cp.wait()
    one = jnp.ones((_L,), jnp.float32)

    def body(i, carry):
        for u in range(_UNROLL):
            idx = e_v[1, pl.ds((i * _UNROLL + u) * _L, _L)]
            plsc.addupdate_scatter(hist_v, [idx], one)
        return carry

    lax.fori_loop(0, nvreg // _UNROLL, body, 0)
    pltpu.sync_copy(hist_v, part_hbm.at[wid])


@functools.partial(
    pl.kernel,
    out_type=jax.ShapeDtypeStruct((_NW, _NP), jnp.float32),
    mesh=_mesh,
    scratch_types=[
        pltpu.VMEM((_NP,), jnp.float32),
        pltpu.VMEM((2, _EBUF), jnp.int32),
        pltpu.VMEM((_NP,), jnp.float32),
        pltpu.SemaphoreType.DMA,
    ],
    compiler_params=_sc_params,
)
def _edge_scatter(edge_hbm, g_hbm, part_hbm, g_v, e_v, acc_v, sem):
    cid = lax.axis_index("c")
    sid = lax.axis_index("s")
    wid = sid * _NC + cid
    base, nvreg = _edge_range(wid)
    cp1 = pltpu.async_copy(g_hbm, g_v, sem)
    cp2 = pltpu.async_copy(edge_hbm.at[:, pl.ds(base, _EBUF)], e_v, sem)
    _zero_vmem(acc_v, _NP)
    cp1.wait()
    cp2.wait()

    def body(i, carry):
        for u in range(_UNROLL):
            off = (i * _UNROLL + u) * _L
            sidx = e_v[0, pl.ds(off, _L)]
            didx = e_v[1, pl.ds(off, _L)]
            vals = plsc.load_gather(g_v, [sidx])
            plsc.addupdate_scatter(acc_v, [didx], vals)
        return carry

    lax.fori_loop(0, nvreg // _UNROLL, body, 0)
    pltpu.sync_copy(acc_v, part_hbm.at[wid])


def _matvec_body(x_ref, w_ref, y_ref):
    y = jnp.dot(x_ref[...], w_ref[...],
                preferred_element_type=jnp.float32)[:, 0]
    y_ref[...] = jnp.concatenate([y, jnp.zeros((_NP - _N,), jnp.float32)])


_matvec = pl.pallas_call(
    _matvec_body,
    out_shape=jax.ShapeDtypeStruct((_NP,), jnp.float32),
)


def _norm_body(degp_ref, y_ref, g_ref):
    deg = jnp.sum(degp_ref[...], axis=0) + 1.0
    g_ref[...] = lax.rsqrt(deg) * y_ref[...]


_norm = pl.pallas_call(
    _norm_body,
    out_shape=jax.ShapeDtypeStruct((_NP,), jnp.float32),
)


def _final_body(accp_ref, degp_ref, y_ref, out_ref):
    acc = jnp.sum(accp_ref[...], axis=0)
    deg = jnp.sum(degp_ref[...], axis=0) + 1.0
    dinv = lax.rsqrt(deg)
    out_ref[...] = dinv * (acc + dinv * y_ref[...])


_final = pl.pallas_call(
    _final_body,
    out_shape=jax.ShapeDtypeStruct((_NP,), jnp.float32),
)


def kernel(x, edge_index, W):
    y = _matvec(x, W)
    deg_part = _hist(edge_index)
    g = _norm(deg_part, y)
    acc_part = _edge_scatter(edge_index, g)
    out = _final(acc_part, deg_part, y)
    return out[:_N]


# speedup vs baseline: 1.0814x; 1.0148x over previous
import functools

import jax
import jax.numpy as jnp
from jax import lax
from jax.experimental import pallas as pl
from jax.experimental.pallas import tpu as pltpu
from jax.experimental.pallas import tpu_sc as plsc

_N = 10000
_E = 320000
_D = 128
_NP = 10240
_NC = 2
_NS = 16
_NW = _NC * _NS
_EB = _E // 128
_EBUF = 79 * 128
_L = 16
_UNROLL = 8
_SPLIT = 40 * 128
_SVREG = _SPLIT // _L

_mesh = plsc.VectorSubcoreMesh(core_axis_name="c", subcore_axis_name="s")
_sc_params = pltpu.CompilerParams(needs_layout_passes=False)


def _edge_range(wid):
    blk_s = (_EB * wid) // _NW
    blk_e = (_EB * (wid + 1)) // _NW
    base = pl.multiple_of(blk_s * 128, 128)
    nvreg = (blk_e - blk_s) * (128 // _L)
    return base, nvreg


def _zero_vmem(ref, n):
    z = jnp.zeros((_L,), jnp.float32)

    def body(i, carry):
        for u in range(8):
            ref[pl.ds((i * 8 + u) * _L, _L)] = z
        return carry

    lax.fori_loop(0, n // (8 * _L), body, 0)


@functools.partial(
    pl.kernel,
    out_type=jax.ShapeDtypeStruct((_NW, _NP), jnp.float32),
    mesh=_mesh,
    scratch_types=[
        pltpu.VMEM((2, _EBUF), jnp.int32),
        pltpu.VMEM((_NP,), jnp.float32),
        pltpu.SemaphoreType.DMA,
        pltpu.SemaphoreType.DMA,
    ],
    compiler_params=_sc_params,
)
def _hist(edge_hbm, part_hbm, e_v, hist_v, sem_a, sem_b):
    cid = lax.axis_index("c")
    sid = lax.axis_index("s")
    wid = sid * _NC + cid
    base, nvreg = _edge_range(wid)
    cp_a = pltpu.async_copy(edge_hbm.at[:, pl.ds(base, _SPLIT)],
                            e_v.at[:, pl.ds(0, _SPLIT)], sem_a)
    cp_b = pltpu.async_copy(
        edge_hbm.at[:, pl.ds(base + _SPLIT, _EBUF - _SPLIT)],
        e_v.at[:, pl.ds(_SPLIT, _EBUF - _SPLIT)], sem_b)
    _zero_vmem(hist_v, _NP)
    cp_a.wait()
    one = jnp.ones((_L,), jnp.float32)

    def body(i, carry):
        for u in range(_UNROLL):
            idx = e_v[1, pl.ds((i * _UNROLL + u) * _L, _L)]
            plsc.addupdate_scatter(hist_v, [idx], one)
        return carry

    lax.fori_loop(0, _SVREG // _UNROLL, body, 0)
    cp_b.wait()
    lax.fori_loop(_SVREG // _UNROLL, nvreg // _UNROLL, body, 0)
    pltpu.sync_copy(hist_v, part_hbm.at[wid])


@functools.partial(
    pl.kernel,
    out_type=jax.ShapeDtypeStruct((_NW, _NP), jnp.float32),
    mesh=_mesh,
    scratch_types=[
        pltpu.VMEM((_NP,), jnp.float32),
        pltpu.VMEM((2, _EBUF), jnp.int32),
        pltpu.VMEM((_NP,), jnp.float32),
        pltpu.SemaphoreType.DMA,
        pltpu.SemaphoreType.DMA,
        pltpu.SemaphoreType.DMA,
    ],
    compiler_params=_sc_params,
)
def _edge_scatter(edge_hbm, g_hbm, part_hbm, g_v, e_v, acc_v,
                  sem_g, sem_a, sem_b):
    cid = lax.axis_index("c")
    sid = lax.axis_index("s")
    wid = sid * _NC + cid
    base, nvreg = _edge_range(wid)
    cp_g = pltpu.async_copy(g_hbm, g_v, sem_g)
    cp_a = pltpu.async_copy(edge_hbm.at[:, pl.ds(base, _SPLIT)],
                            e_v.at[:, pl.ds(0, _SPLIT)], sem_a)
    cp_b = pltpu.async_copy(
        edge_hbm.at[:, pl.ds(base + _SPLIT, _EBUF - _SPLIT)],
        e_v.at[:, pl.ds(_SPLIT, _EBUF - _SPLIT)], sem_b)
    _zero_vmem(acc_v, _NP)
    cp_g.wait()
    cp_a.wait()

    def body(i, carry):
        for u in range(_UNROLL):
            off = (i * _UNROLL + u) * _L
            sidx = e_v[0, pl.ds(off, _L)]
            didx = e_v[1, pl.ds(off, _L)]
            vals = plsc.load_gather(g_v, [sidx])
            plsc.addupdate_scatter(acc_v, [didx], vals)
        return carry

    lax.fori_loop(0, _SVREG // _UNROLL, body, 0)
    cp_b.wait()
    lax.fori_loop(_SVREG // _UNROLL, nvreg // _UNROLL, body, 0)
    pltpu.sync_copy(acc_v, part_hbm.at[wid])


def _matvec_body(x_ref, w_ref, y_ref):
    y = jnp.dot(x_ref[...], w_ref[...],
                preferred_element_type=jnp.float32)[:, 0]
    y_ref[...] = jnp.concatenate([y, jnp.zeros((_NP - _N,), jnp.float32)])


_matvec = pl.pallas_call(
    _matvec_body,
    out_shape=jax.ShapeDtypeStruct((_NP,), jnp.float32),
)


def _norm_body(degp_ref, y_ref, g_ref):
    deg = jnp.sum(degp_ref[...], axis=0) + 1.0
    g_ref[...] = lax.rsqrt(deg) * y_ref[...]


_norm = pl.pallas_call(
    _norm_body,
    out_shape=jax.ShapeDtypeStruct((_NP,), jnp.float32),
)


def _final_body(accp_ref, degp_ref, y_ref, out_ref):
    acc = jnp.sum(accp_ref[...], axis=0)
    deg = jnp.sum(degp_ref[...], axis=0) + 1.0
    dinv = lax.rsqrt(deg)
    out_ref[...] = dinv * (acc + dinv * y_ref[...])


_final = pl.pallas_call(
    _final_body,
    out_shape=jax.ShapeDtypeStruct((_NP,), jnp.float32),
)


def kernel(x, edge_index, W):
    y = _matvec(x, W)
    deg_part = _hist(edge_index)
    g = _norm(deg_part, y)
    acc_part = _edge_scatter(edge_index, g)
    out = _final(acc_part, deg_part, y)
    return out[:_N]


# speedup vs baseline: 1.1011x; 1.0182x over previous
import functools

import jax
import jax.numpy as jnp
from jax import lax
from jax.experimental import pallas as pl
from jax.experimental.pallas import tpu as pltpu
from jax.experimental.pallas import tpu_sc as plsc

_N = 10000
_E = 320000
_D = 128
_NP = 10240
_NC = 2
_NS = 16
_NW = _NC * _NS
_EB = _E // 128
_EBUF = 79 * 128
_L = 16
_UNROLL = 8
_SPLIT = 40 * 128
_SVREG = _SPLIT // _L

_mesh = plsc.VectorSubcoreMesh(core_axis_name="c", subcore_axis_name="s")
_sc_params = pltpu.CompilerParams(needs_layout_passes=False)


def _edge_range(wid):
    blk_s = (_EB * wid) // _NW
    blk_e = (_EB * (wid + 1)) // _NW
    base = pl.multiple_of(blk_s * 128, 128)
    nvreg = (blk_e - blk_s) * (128 // _L)
    return base, nvreg


def _zero_vmem(ref, n):
    z = jnp.zeros((_L,), jnp.float32)

    def body(i, carry):
        for u in range(8):
            ref[pl.ds((i * 8 + u) * _L, _L)] = z
        return carry

    lax.fori_loop(0, n // (8 * _L), body, 0)


@functools.partial(
    pl.kernel,
    out_type=jax.ShapeDtypeStruct((_NW, _NP), jnp.float32),
    mesh=_mesh,
    scratch_types=[
        pltpu.VMEM((2, _EBUF), jnp.int32),
        pltpu.VMEM((_NP,), jnp.float32),
        pltpu.SemaphoreType.DMA,
        pltpu.SemaphoreType.DMA,
    ],
    compiler_params=_sc_params,
)
def _hist(edge_hbm, part_hbm, e_v, hist_v, sem_a, sem_b):
    cid = lax.axis_index("c")
    sid = lax.axis_index("s")
    wid = sid * _NC + cid
    base, nvreg = _edge_range(wid)
    cp_a = pltpu.async_copy(edge_hbm.at[:, pl.ds(base, _SPLIT)],
                            e_v.at[:, pl.ds(0, _SPLIT)], sem_a)
    cp_b = pltpu.async_copy(
        edge_hbm.at[:, pl.ds(base + _SPLIT, _EBUF - _SPLIT)],
        e_v.at[:, pl.ds(_SPLIT, _EBUF - _SPLIT)], sem_b)
    _zero_vmem(hist_v, _NP)
    cp_a.wait()
    one = jnp.ones((_L,), jnp.float32)

    def body(i, carry):
        for u in range(_UNROLL):
            idx = e_v[1, pl.ds((i * _UNROLL + u) * _L, _L)]
            plsc.addupdate_scatter(hist_v, [idx], one)
        return carry

    lax.fori_loop(0, _SVREG // _UNROLL, body, 0)
    cp_b.wait()
    lax.fori_loop(_SVREG // _UNROLL, nvreg // _UNROLL, body, 0)
    pltpu.sync_copy(hist_v, part_hbm.at[wid])


@functools.partial(
    pl.kernel,
    out_type=jax.ShapeDtypeStruct((_NW, _NP), jnp.float32),
    mesh=_mesh,
    scratch_types=[
        pltpu.VMEM((_NP,), jnp.float32),
        pltpu.VMEM((2, _EBUF), jnp.int32),
        pltpu.VMEM((_NP,), jnp.float32),
        pltpu.SemaphoreType.DMA,
        pltpu.SemaphoreType.DMA,
        pltpu.SemaphoreType.DMA,
    ],
    compiler_params=_sc_params,
)
def _edge_scatter(edge_hbm, g_hbm, part_hbm, g_v, e_v, acc_v,
                  sem_g, sem_a, sem_b):
    cid = lax.axis_index("c")
    sid = lax.axis_index("s")
    wid = sid * _NC + cid
    base, nvreg = _edge_range(wid)
    cp_g = pltpu.async_copy(g_hbm, g_v, sem_g)
    cp_a = pltpu.async_copy(edge_hbm.at[:, pl.ds(base, _SPLIT)],
                            e_v.at[:, pl.ds(0, _SPLIT)], sem_a)
    cp_b = pltpu.async_copy(
        edge_hbm.at[:, pl.ds(base + _SPLIT, _EBUF - _SPLIT)],
        e_v.at[:, pl.ds(_SPLIT, _EBUF - _SPLIT)], sem_b)
    _zero_vmem(acc_v, _NP)
    cp_g.wait()
    cp_a.wait()

    def body(i, carry):
        for u in range(_UNROLL):
            off = (i * _UNROLL + u) * _L
            sidx = e_v[0, pl.ds(off, _L)]
            didx = e_v[1, pl.ds(off, _L)]
            vals = plsc.load_gather(g_v, [sidx])
            plsc.addupdate_scatter(acc_v, [didx], vals)
        return carry

    lax.fori_loop(0, _SVREG // _UNROLL, body, 0)
    cp_b.wait()
    lax.fori_loop(_SVREG // _UNROLL, nvreg // _UNROLL, body, 0)
    pltpu.sync_copy(acc_v, part_hbm.at[wid])


def _matvec_body(x_ref, w_ref, y_ref):
    y = jnp.dot(x_ref[...], w_ref[...],
                preferred_element_type=jnp.float32)[:, 0]
    y_ref[...] = jnp.concatenate([y, jnp.zeros((_NP - _N,), jnp.float32)])


_matvec = pl.pallas_call(
    _matvec_body,
    out_shape=jax.ShapeDtypeStruct((_NP,), jnp.float32),
)


def _norm_body(degp_ref, y_ref, g_ref, dinv_ref, h_ref):
    deg = jnp.sum(degp_ref[...], axis=0) + 1.0
    dinv = lax.rsqrt(deg)
    g = dinv * y_ref[...]
    g_ref[...] = g
    dinv_ref[...] = dinv
    h_ref[...] = dinv * g


_norm = pl.pallas_call(
    _norm_body,
    out_shape=(
        jax.ShapeDtypeStruct((_NP,), jnp.float32),
        jax.ShapeDtypeStruct((_NP,), jnp.float32),
        jax.ShapeDtypeStruct((_NP,), jnp.float32),
    ),
)


def _final_body(accp_ref, dinv_ref, h_ref, out_ref):
    acc = jnp.sum(accp_ref[...], axis=0)
    out = dinv_ref[...] * acc + h_ref[...]
    out_ref[...] = out[:_N]


_final = pl.pallas_call(
    _final_body,
    out_shape=jax.ShapeDtypeStruct((_N,), jnp.float32),
)


def kernel(x, edge_index, W):
    y = _matvec(x, W)
    deg_part = _hist(edge_index)
    g, dinv, h = _norm(deg_part, y)
    acc_part = _edge_scatter(edge_index, g)
    return _final(acc_part, dinv, h)


# speedup vs baseline: 1.2597x; 1.1440x over previous
import functools

import jax
import jax.numpy as jnp
from jax import lax
from jax.experimental import pallas as pl
from jax.experimental.pallas import tpu as pltpu
from jax.experimental.pallas import tpu_sc as plsc

_N = 10000
_E = 320000
_D = 128
_NP = 10240
_NC = 2
_NS = 16
_NW = _NC * _NS
_EB = _E // 128
_EBUF = 79 * 128
_L = 16
_UNROLL = 8
_SPLIT = 40 * 128
_SVREG = _SPLIT // _L

_mesh = plsc.VectorSubcoreMesh(core_axis_name="c", subcore_axis_name="s")
_sc_params = pltpu.CompilerParams(needs_layout_passes=False)


def _edge_range(wid):
    blk_s = (_EB * wid) // _NW
    blk_e = (_EB * (wid + 1)) // _NW
    base = pl.multiple_of(blk_s * 128, 128)
    nvreg = (blk_e - blk_s) * (128 // _L)
    return base, nvreg


def _zero_vmem(ref, n):
    z = jnp.zeros((_L,), jnp.float32)

    def body(i, carry):
        for u in range(8):
            ref[pl.ds((i * 8 + u) * _L, _L)] = z
        return carry

    lax.fori_loop(0, n // (8 * _L), body, 0)


@functools.partial(
    pl.kernel,
    out_type=jax.ShapeDtypeStruct((_NW, _NP), jnp.float32),
    mesh=_mesh,
    scratch_types=[
        pltpu.VMEM((2, _EBUF), jnp.int32),
        pltpu.VMEM((_NP,), jnp.float32),
        pltpu.SemaphoreType.DMA,
        pltpu.SemaphoreType.DMA,
    ],
    compiler_params=_sc_params,
)
def _hist(edge_hbm, part_hbm, e_v, hist_v, sem_a, sem_b):
    cid = lax.axis_index("c")
    sid = lax.axis_index("s")
    wid = sid * _NC + cid
    base, nvreg = _edge_range(wid)
    cp_a = pltpu.async_copy(edge_hbm.at[:, pl.ds(base, _SPLIT)],
                            e_v.at[:, pl.ds(0, _SPLIT)], sem_a)
    cp_b = pltpu.async_copy(
        edge_hbm.at[:, pl.ds(base + _SPLIT, _EBUF - _SPLIT)],
        e_v.at[:, pl.ds(_SPLIT, _EBUF - _SPLIT)], sem_b)
    _zero_vmem(hist_v, _NP)
    cp_a.wait()
    one = jnp.ones((_L,), jnp.float32)

    def body(i):
        idx = e_v[1, pl.ds(i * _L, _L)]
        plsc.addupdate_scatter(hist_v, [idx], one)

    plsc.parallel_loop(0, _SVREG, unroll=_UNROLL)(body)
    cp_b.wait()
    plsc.parallel_loop(_SVREG, nvreg, unroll=_UNROLL)(body)
    pltpu.sync_copy(hist_v, part_hbm.at[wid])


@functools.partial(
    pl.kernel,
    out_type=jax.ShapeDtypeStruct((_NW, _NP), jnp.float32),
    mesh=_mesh,
    scratch_types=[
        pltpu.VMEM((_NP,), jnp.float32),
        pltpu.VMEM((2, _EBUF), jnp.int32),
        pltpu.VMEM((_NP,), jnp.float32),
        pltpu.SemaphoreType.DMA,
        pltpu.SemaphoreType.DMA,
        pltpu.SemaphoreType.DMA,
    ],
    compiler_params=_sc_params,
)
def _edge_scatter(edge_hbm, g_hbm, part_hbm, g_v, e_v, acc_v,
                  sem_g, sem_a, sem_b):
    cid = lax.axis_index("c")
    sid = lax.axis_index("s")
    wid = sid * _NC + cid
    base, nvreg = _edge_range(wid)
    cp_g = pltpu.async_copy(g_hbm, g_v, sem_g)
    cp_a = pltpu.async_copy(edge_hbm.at[:, pl.ds(base, _SPLIT)],
                            e_v.at[:, pl.ds(0, _SPLIT)], sem_a)
    cp_b = pltpu.async_copy(
        edge_hbm.at[:, pl.ds(base + _SPLIT, _EBUF - _SPLIT)],
        e_v.at[:, pl.ds(_SPLIT, _EBUF - _SPLIT)], sem_b)
    _zero_vmem(acc_v, _NP)
    cp_g.wait()
    cp_a.wait()

    def body(i):
        off = i * _L
        sidx = e_v[0, pl.ds(off, _L)]
        didx = e_v[1, pl.ds(off, _L)]
        vals = plsc.load_gather(g_v, [sidx])
        plsc.addupdate_scatter(acc_v, [didx], vals)

    plsc.parallel_loop(0, _SVREG, unroll=_UNROLL)(body)
    cp_b.wait()
    plsc.parallel_loop(_SVREG, nvreg, unroll=_UNROLL)(body)
    pltpu.sync_copy(acc_v, part_hbm.at[wid])


def _matvec_body(x_ref, w_ref, y_ref):
    y = jnp.dot(x_ref[...], w_ref[...],
                preferred_element_type=jnp.float32)[:, 0]
    y_ref[...] = jnp.concatenate([y, jnp.zeros((_NP - _N,), jnp.float32)])


_matvec = pl.pallas_call(
    _matvec_body,
    out_shape=jax.ShapeDtypeStruct((_NP,), jnp.float32),
)


def _norm_body(degp_ref, y_ref, g_ref, dinv_ref, h_ref):
    deg = jnp.sum(degp_ref[...], axis=0) + 1.0
    dinv = lax.rsqrt(deg)
    g = dinv * y_ref[...]
    g_ref[...] = g
    dinv_ref[...] = dinv
    h_ref[...] = dinv * g


_norm = pl.pallas_call(
    _norm_body,
    out_shape=(
        jax.ShapeDtypeStruct((_NP,), jnp.float32),
        jax.ShapeDtypeStruct((_NP,), jnp.float32),
        jax.ShapeDtypeStruct((_NP,), jnp.float32),
    ),
)


def _final_body(accp_ref, dinv_ref, h_ref, out_ref):
    acc = jnp.sum(accp_ref[...], axis=0)
    out = dinv_ref[...] * acc + h_ref[...]
    out_ref[...] = out[:_N]


_final = pl.pallas_call(
    _final_body,
    out_shape=jax.ShapeDtypeStruct((_N,), jnp.float32),
)


def kernel(x, edge_index, W):
    y = _matvec(x, W)
    deg_part = _hist(edge_index)
    g, dinv, h = _norm(deg_part, y)
    acc_part = _edge_scatter(edge_index, g)
    return _final(acc_part, dinv, h)
